# baseline (speedup 1.0000x reference)
import functools

import jax
import jax.numpy as jnp
from jax import lax
from jax.experimental import pallas as pl
from jax.experimental.pallas import tpu as pltpu
from jax.experimental.pallas import tpu_sc as plsc

LEN = 128
NW = 32


def _gather_channel(x, ch):
    T, C, H, W = x.shape
    per_w = LEN // NW
    a, b = 2 * (T - 1), 2 * (LEN - 1)

    mesh = plsc.VectorSubcoreMesh(core_axis_name="c", subcore_axis_name="s")

    @functools.partial(
        pl.kernel,
        out_type=jax.ShapeDtypeStruct((LEN, H, W), x.dtype),
        mesh=mesh,
        scratch_types=[
            pltpu.VMEM((2, H, W), x.dtype),
            pltpu.SemaphoreType.DMA,
            pltpu.SemaphoreType.DMA,
            pltpu.SemaphoreType.DMA,
            pltpu.SemaphoreType.DMA,
        ],
        name=f"sc_gather_c{ch}",
    )
    def k(x_hbm, out_hbm, buf, si0, si1, so0, so1):
        wid = lax.axis_index("s") * 2 + lax.axis_index("c")
        base = wid * per_w
        sin = (si0, si1)
        sout = (so0, so1)

        def start_in(q, slot):
            o = base + q
            src = (o * a + (LEN - 1)) // b
            return pltpu.async_copy(x_hbm.at[src, ch], buf.at[slot], sin[slot])

        def start_out(q, slot):
            return pltpu.async_copy(buf.at[slot], out_hbm.at[base + q], sout[slot])

        in_cp = [None, None]
        out_cp = [None, None]
        in_cp[0] = start_in(0, 0)
        for q in range(per_w):
            slot = q % 2
            nxt = (q + 1) % 2
            if q + 1 < per_w:
                if q >= 1:
                    out_cp[nxt].wait()
                in_cp[nxt] = start_in(q + 1, nxt)
            in_cp[slot].wait()
            out_cp[slot] = start_out(q, slot)
        out_cp[0].wait()
        out_cp[1].wait()

    return k(x)


def kernel(x):
    C = x.shape[1]
    parts = [_gather_channel(x, ch) for ch in range(C)]
    return jnp.stack(parts, axis=1)
